# Initial kernel scaffold; baseline (speedup 1.0000x reference)
import jax
import jax.numpy as jnp
from jax.experimental import pallas as pl


def kernel(x, W):
    raise NotImplementedError("write your pallas kernel here")


# speedup vs baseline: 1.2558x; 1.2558x over previous
import functools
import jax
import jax.numpy as jnp
from jax import lax
from jax.experimental import pallas as pl
from jax.experimental.pallas import tpu as pltpu
from jax.experimental.pallas import tpu_sc as plsc

B, T, D = 4, 4096, 2048
K = T // 2
ROWS, LANES = 32, 128
KROWS = K // LANES


_BT = 1024


def _scores_kernel(x_ref, w_ref, o_ref):
    o_ref[0] = lax.dot_general(
        w_ref[...], x_ref[...], (((1,), (1,)), ((), ())),
        preferred_element_type=jnp.float32)


def _scores(x2d, w2d):
    n = x2d.shape[0]
    return pl.pallas_call(
        _scores_kernel,
        grid=(n // _BT,),
        in_specs=[
            pl.BlockSpec((_BT, D), lambda i: (i, 0)),
            pl.BlockSpec((1, D), lambda i: (0, 0)),
        ],
        out_specs=pl.BlockSpec((1, 1, _BT), lambda i: (i, 0, 0)),
        out_shape=jax.ShapeDtypeStruct((n // _BT, 1, _BT), jnp.float32),
    )(x2d, w2d)


def _topk_kernel(s_ref, i_ref, f_ref):
    b = pl.program_id(0)
    s2 = s_ref[0]
    rows = lax.broadcasted_iota(jnp.int32, (ROWS, LANES), 0)
    lanes = lax.broadcasted_iota(jnp.int32, (ROWS, LANES), 1)
    i2 = rows * LANES + lanes

    def partner(v, d):
        if d < LANES:
            m = (lanes & d) == 0
            return jnp.where(m, pltpu.roll(v, LANES - d, 1),
                             pltpu.roll(v, d, 1)), m
        r = d // LANES
        m = (rows & r) == 0
        return jnp.where(m, pltpu.roll(v, ROWS - r, 0),
                         pltpu.roll(v, r, 0)), m

    kblock = 2
    while kblock <= T:
        d = kblock // 2
        while d >= 1:
            sp, low = partner(s2, d)
            ip, _ = partner(i2, d)
            bfr = (s2 > sp) | ((s2 == sp) & (i2 < ip))
            if kblock < T:
                keep = bfr ^ (~low) ^ (((rows * LANES + lanes) & kblock) != 0)
            else:
                keep = bfr ^ (~low)
            s2 = jnp.where(keep, s2, sp)
            i2 = jnp.where(keep, i2, ip)
            d //= 2
        kblock *= 2

    i_ref[0] = i2[:KROWS]
    f_ref[0] = i2[:KROWS] + b * T


def _topk(scores3):
    return pl.pallas_call(
        _topk_kernel,
        grid=(B,),
        in_specs=[pl.BlockSpec((1, ROWS, LANES), lambda b: (b, 0, 0))],
        out_specs=[
            pl.BlockSpec((1, KROWS, LANES), lambda b: (b, 0, 0)),
            pl.BlockSpec((1, KROWS, LANES), lambda b: (b, 0, 0)),
        ],
        out_shape=[
            jax.ShapeDtypeStruct((B, KROWS, LANES), jnp.int32),
            jax.ShapeDtypeStruct((B, KROWS, LANES), jnp.int32),
        ],
    )(scores3)


_NC, _NS = 2, 16
_NW = _NC * _NS
_RPW = (B * K) // _NW
_CH = 16
_NCHUNK = _RPW // _CH


def _gather_body(idx_hbm, x_hbm, out_hbm, idx_v, buf0, buf1, gsem):
    wid = lax.axis_index("s") * _NC + lax.axis_index("c")
    base = wid * _NCHUNK
    pltpu.sync_copy(idx_hbm.at[pl.ds(base, _NCHUNK)], idx_v)
    bufs = (buf0, buf1)

    def start_gather(c):
        return pltpu.async_copy(x_hbm.at[idx_v.at[c]], bufs[c % 2], gsem)

    g = [None] * _NCHUNK
    g[0] = start_gather(0)
    for c in range(_NCHUNK):
        if c + 1 < _NCHUNK:
            g[c + 1] = start_gather(c + 1)
        g[c].wait()
        pltpu.sync_copy(bufs[c % 2],
                        out_hbm.at[pl.ds(wid * _RPW + c * _CH, _CH)])


def _gather(idx2d, x2d):
    mesh = plsc.VectorSubcoreMesh(core_axis_name="c", subcore_axis_name="s")
    f = pl.kernel(
        _gather_body,
        out_type=jax.ShapeDtypeStruct((B * K, D), jnp.float32),
        mesh=mesh,
        scratch_types=[
            pltpu.VMEM((_NCHUNK, _CH), jnp.int32),
            pltpu.VMEM((_CH, D), jnp.float32),
            pltpu.VMEM((_CH, D), jnp.float32),
            pltpu.SemaphoreType.DMA,
        ],
    )
    return f(idx2d, x2d)


def kernel(x, W):
    x2d = x.reshape(B * T, D)
    scores = _scores(x2d, W.reshape(1, D)).reshape(B, T)
    idx3, flat3 = _topk(scores.reshape(B, ROWS, LANES))
    indices = idx3.reshape(B, K)
    flat = flat3.reshape((B * K) // _CH, _CH)
    selected = _gather(flat, x2d).reshape(B, K, D)
    return (selected, indices, scores)
